# Initial kernel scaffold; baseline (speedup 1.0000x reference)
import jax
import jax.numpy as jnp
from jax.experimental import pallas as pl


def kernel(x, weight):
    raise NotImplementedError("write your pallas kernel here")


# speedup vs baseline: 3.2658x; 3.2658x over previous
import functools

import jax
import jax.numpy as jnp
from jax import lax
from jax.experimental import pallas as pl
from jax.experimental.pallas import tpu as pltpu
from jax.experimental.pallas import tpu_sc as plsc

_D = 128
_BATCH = 16384
_HIST = 50
_TOTAL = _BATCH * _HIST
_IDXROW = 128
_NROWS = _TOTAL // _IDXROW
_NC, _NS = 2, 16
_NW = _NC * _NS
_ROWS_PER_W = _NROWS // _NW
_K = 2
_CHUNKS = _ROWS_PER_W // _K

_mesh = plsc.VectorSubcoreMesh(core_axis_name="c", subcore_axis_name="s")


@functools.partial(
    pl.kernel,
    out_type=jax.ShapeDtypeStruct((_TOTAL, _D), jnp.float32),
    mesh=_mesh,
    scratch_types=[
        pltpu.VMEM((_ROWS_PER_W, _IDXROW), jnp.int32),
        pltpu.VMEM((_K * _IDXROW, _D), jnp.float32),
        pltpu.SemaphoreType.DMA,
    ],
)
def _emb(x_hbm, w_hbm, out_hbm, idx_v, rows_v, sem):
    wid = lax.axis_index("s") * _NC + lax.axis_index("c")
    base = wid * _ROWS_PER_W
    pltpu.sync_copy(x_hbm.at[pl.ds(base, _ROWS_PER_W)], idx_v)

    @pl.loop(0, _CHUNKS)
    def _chunk(c):
        r0 = c * _K
        copies = []
        for j in range(_K):
            copies.append(
                pltpu.async_copy(
                    w_hbm.at[idx_v.at[r0 + j]],
                    rows_v.at[pl.ds(j * _IDXROW, _IDXROW)],
                    sem,
                )
            )
        for cp in copies:
            cp.wait()
        pltpu.sync_copy(
            rows_v,
            out_hbm.at[pl.ds((base + r0) * _IDXROW, _K * _IDXROW)],
        )


def kernel(x, weight):
    xr = x.reshape(_TOTAL).astype(jnp.int32).reshape(_NROWS, _IDXROW)
    out = _emb(xr, weight)
    return out.reshape(_BATCH, _HIST, _D)


# speedup vs baseline: 3.4532x; 1.0574x over previous
import functools

import jax
import jax.numpy as jnp
from jax import lax
from jax.experimental import pallas as pl
from jax.experimental.pallas import tpu as pltpu
from jax.experimental.pallas import tpu_sc as plsc

_D = 128
_BATCH = 16384
_HIST = 50
_TOTAL = _BATCH * _HIST
_IDXROW = 128
_NROWS = _TOTAL // _IDXROW
_NC, _NS = 2, 16
_NW = _NC * _NS
_ROWS_PER_W = _NROWS // _NW
_K = 2
_CHUNKS = _ROWS_PER_W // _K
_NBUF = 2
_CROWS = _K * _IDXROW

_mesh = plsc.VectorSubcoreMesh(core_axis_name="c", subcore_axis_name="s")


@functools.partial(
    pl.kernel,
    out_type=jax.ShapeDtypeStruct((_TOTAL, _D), jnp.float32),
    mesh=_mesh,
    scratch_types=[
        pltpu.VMEM((_ROWS_PER_W, _IDXROW), jnp.int32),
        pltpu.VMEM((_NBUF, _CROWS, _D), jnp.float32),
        pltpu.SemaphoreType.DMA,
        pltpu.SemaphoreType.DMA,
    ],
)
def _emb(x_hbm, w_hbm, out_hbm, idx_v, rows_v, sem0, sem1):
    wid = lax.axis_index("s") * _NC + lax.axis_index("c")
    base = wid * _ROWS_PER_W
    sems = (sem0, sem1)
    pltpu.sync_copy(x_hbm.at[pl.ds(base, _ROWS_PER_W)], idx_v)

    def _fire(c, buf):
        r0 = c * _K
        for j in range(_K):
            pltpu.async_copy(
                w_hbm.at[idx_v.at[r0 + j]],
                rows_v.at[buf].at[pl.ds(j * _IDXROW, _IDXROW)],
                sems[buf],
            )

    _fire(0, 0)

    @pl.loop(0, _CHUNKS, step=_NBUF)
    def _outer(cc):
        for b in range(_NBUF):
            c = cc + b

            @pl.when(c + 1 < _CHUNKS)
            def _():
                _fire(c + 1, (b + 1) % _NBUF)

            pltpu.make_async_copy(
                out_hbm.at[pl.ds(0, _CROWS)], rows_v.at[b], sems[b]
            ).wait()
            pltpu.sync_copy(
                rows_v.at[b],
                out_hbm.at[pl.ds((base + c * _K) * _IDXROW, _CROWS)],
            )


def kernel(x, weight):
    xr = x.reshape(_TOTAL).astype(jnp.int32).reshape(_NROWS, _IDXROW)
    out = _emb(xr, weight)
    return out.reshape(_BATCH, _HIST, _D)


# speedup vs baseline: 3.4546x; 1.0004x over previous
import functools

import jax
import jax.numpy as jnp
from jax import lax
from jax.experimental import pallas as pl
from jax.experimental.pallas import tpu as pltpu
from jax.experimental.pallas import tpu_sc as plsc

_D = 128
_BATCH = 16384
_HIST = 50
_TOTAL = _BATCH * _HIST
_IDXROW = 128
_NROWS = _TOTAL // _IDXROW
_NC, _NS = 2, 16
_NW = _NC * _NS
_ROWS_PER_W = _NROWS // _NW
_K = 2
_CHUNKS = _ROWS_PER_W // _K
_NBUF = 3
_CROWS = _K * _IDXROW
_MAIN = (_CHUNKS - 1) // _NBUF * _NBUF

_mesh = plsc.VectorSubcoreMesh(core_axis_name="c", subcore_axis_name="s")


@functools.partial(
    pl.kernel,
    out_type=jax.ShapeDtypeStruct((_TOTAL, _D), jnp.float32),
    mesh=_mesh,
    scratch_types=[
        pltpu.VMEM((_ROWS_PER_W, _IDXROW), jnp.int32),
        pltpu.VMEM((_NBUF, _CROWS, _D), jnp.float32),
        pltpu.SemaphoreType.DMA,
        pltpu.SemaphoreType.DMA,
        pltpu.SemaphoreType.DMA,
    ],
)
def _emb(x_hbm, w_hbm, out_hbm, idx_v, rows_v, sem0, sem1, sem2):
    wid = lax.axis_index("s") * _NC + lax.axis_index("c")
    base = wid * _ROWS_PER_W
    sems = (sem0, sem1, sem2)
    pltpu.sync_copy(x_hbm.at[pl.ds(base, _ROWS_PER_W)], idx_v)

    def _fire(c, buf):
        r0 = c * _K
        for j in range(_K):
            pltpu.async_copy(
                w_hbm.at[idx_v.at[r0 + j]],
                rows_v.at[buf].at[pl.ds(j * _IDXROW, _IDXROW)],
                sems[buf],
            )

    def _drain_store(c, b):
        pltpu.make_async_copy(
            out_hbm.at[pl.ds(0, _CROWS)], rows_v.at[b], sems[b]
        ).wait()
        pltpu.sync_copy(
            rows_v.at[b],
            out_hbm.at[pl.ds((base + c * _K) * _IDXROW, _CROWS)],
        )

    for b in range(_NBUF):
        _fire(b, b)

    @pl.loop(0, _MAIN, step=_NBUF)
    def _outer(cc):
        for b in range(_NBUF):
            c = cc + b
            _drain_store(c, b)

            @pl.when(c + _NBUF < _CHUNKS)
            def _():
                _fire(c + _NBUF, b)

    for c in range(_MAIN, _CHUNKS):
        _drain_store(c, c % _NBUF)


def kernel(x, weight):
    xr = x.reshape(_TOTAL).astype(jnp.int32).reshape(_NROWS, _IDXROW)
    out = _emb(xr, weight)
    return out.reshape(_BATCH, _HIST, _D)


# speedup vs baseline: 6.3895x; 1.8496x over previous
import functools

import jax
import jax.numpy as jnp
from jax import lax
from jax.experimental import pallas as pl
from jax.experimental.pallas import tpu as pltpu
from jax.experimental.pallas import tpu_sc as plsc

_D = 128
_BATCH = 16384
_HIST = 50
_NC, _NS = 2, 16
_NW = _NC * _NS
_BPW = _BATCH // _NW
_G = 2
_CHUNKS = _BPW // _G
_NBUF = 3
_MAIN = -(-(_CHUNKS - _NBUF) // _NBUF) * _NBUF

_mesh = plsc.VectorSubcoreMesh(core_axis_name="c", subcore_axis_name="s")


@functools.partial(
    pl.kernel,
    out_type=jax.ShapeDtypeStruct((_BATCH, _HIST, _D), jnp.float32),
    mesh=_mesh,
    scratch_types=[
        pltpu.VMEM((_BPW, _HIST), jnp.int32),
        pltpu.VMEM((_NBUF, _G, _HIST, _D), jnp.float32),
        pltpu.SemaphoreType.DMA,
        pltpu.SemaphoreType.DMA,
        pltpu.SemaphoreType.DMA,
    ],
)
def _emb(x_hbm, w_hbm, out_hbm, idx_v, rows_v, sem0, sem1, sem2):
    wid = lax.axis_index("s") * _NC + lax.axis_index("c")
    base = wid * _BPW
    sems = (sem0, sem1, sem2)
    pltpu.sync_copy(x_hbm.at[pl.ds(base, _BPW)], idx_v)

    def _fire(c, buf):
        r0 = c * _G
        for j in range(_G):
            pltpu.async_copy(
                w_hbm.at[idx_v.at[r0 + j]],
                rows_v.at[buf].at[j],
                sems[buf],
            )

    def _drain_store(c, b):
        for j in range(_G):
            pltpu.make_async_copy(
                out_hbm.at[0], rows_v.at[b].at[j], sems[b]
            ).wait()
        pltpu.sync_copy(
            rows_v.at[b],
            out_hbm.at[pl.ds(base + c * _G, _G)],
        )

    for b in range(_NBUF):
        _fire(b, b)

    @pl.loop(0, _MAIN, step=_NBUF)
    def _outer(cc):
        for b in range(_NBUF):
            c = cc + b
            _drain_store(c, b)

            @pl.when(c + _NBUF < _CHUNKS)
            def _():
                _fire(c + _NBUF, b)

    for c in range(_MAIN, _CHUNKS):
        _drain_store(c, c % _NBUF)


def kernel(x, weight):
    return _emb(x.astype(jnp.int32), weight)


# speedup vs baseline: 6.3913x; 1.0003x over previous
import functools

import jax
import jax.numpy as jnp
from jax import lax
from jax.experimental import pallas as pl
from jax.experimental.pallas import tpu as pltpu
from jax.experimental.pallas import tpu_sc as plsc

_D = 128
_BATCH = 16384
_HIST = 50
_NC, _NS = 2, 16
_NW = _NC * _NS
_BPW = _BATCH // _NW
_G = 2
_CHUNKS = _BPW // _G
_NBUF = 3
_MAIN = -(-(_CHUNKS - _NBUF) // _NBUF) * _NBUF

_mesh = plsc.VectorSubcoreMesh(core_axis_name="c", subcore_axis_name="s")


@functools.partial(
    pl.kernel,
    out_type=jax.ShapeDtypeStruct((_BATCH, _HIST, _D), jnp.float32),
    mesh=_mesh,
    compiler_params=pltpu.CompilerParams(use_tc_tiling_on_sc=True),
    scratch_types=[
        pltpu.VMEM((_BPW, _HIST), jnp.int32),
        pltpu.VMEM((_NBUF, _G, _HIST, _D), jnp.float32),
        pltpu.SemaphoreType.DMA,
        pltpu.SemaphoreType.DMA,
        pltpu.SemaphoreType.DMA,
    ],
)
def _emb(x_hbm, w_hbm, out_hbm, idx_v, rows_v, sem0, sem1, sem2):
    wid = lax.axis_index("s") * _NC + lax.axis_index("c")
    base = wid * _BPW
    sems = (sem0, sem1, sem2)
    pltpu.sync_copy(x_hbm.at[pl.ds(base, _BPW)], idx_v)

    def _fire(c, buf):
        r0 = c * _G
        for j in range(_G):
            pltpu.async_copy(
                w_hbm.at[idx_v.at[r0 + j]],
                rows_v.at[buf].at[j],
                sems[buf],
            )

    def _drain_store(c, b):
        for j in range(_G):
            pltpu.make_async_copy(
                out_hbm.at[0], rows_v.at[b].at[j], sems[b]
            ).wait()
        pltpu.sync_copy(
            rows_v.at[b],
            out_hbm.at[pl.ds(base + c * _G, _G)],
        )

    for b in range(_NBUF):
        _fire(b, b)

    @pl.loop(0, _MAIN, step=_NBUF)
    def _outer(cc):
        for b in range(_NBUF):
            c = cc + b
            _drain_store(c, b)

            @pl.when(c + _NBUF < _CHUNKS)
            def _():
                _fire(c + _NBUF, b)

    for c in range(_MAIN, _CHUNKS):
        _drain_store(c, c % _NBUF)


def kernel(x, weight):
    return _emb(x.astype(jnp.int32), weight)
